# baseline (speedup 1.0000x reference)
import functools

import jax
import jax.numpy as jnp
from jax import lax
from jax.experimental import pallas as pl
from jax.experimental.pallas import tpu as pltpu
from jax.experimental.pallas import tpu_sc as plsc

B = 1024
L = 77
VOCAB = 49408
MAXLEN = 77
D = 1024

LP = 80
NC = 2
NS = 16
NW = NC * NS
BE_W = B // NW
IDS_W = BE_W * LP
C = 8
CPB = 10
NCHUNK = BE_W * CPB
NBUF = 5
KSTEP = 10
DL = D // 16


def _chunk_geom(k):
    ci = k % CPB
    cw = C if ci < CPB - 1 else L - (CPB - 1) * C
    return ci, cw


def _sc_body(tok80, pos80, tok_tab, pos_tab, out,
             tix, pix, ptab, trows0, trows1, trows2, trows3, trows4,
             gsem0, gsem1, gsem2, gsem3, gsem4,
             ssem0, ssem1, ssem2, ssem3, ssem4):
    wid = lax.axis_index("s") * NC + lax.axis_index("c")
    base_be = wid * BE_W
    pltpu.sync_copy(tok80.at[wid], tix)
    pltpu.sync_copy(pos80.at[wid], pix.at[pl.ds(0, IDS_W)])
    pltpu.sync_copy(pos_tab, ptab)

    trows = (trows0, trows1, trows2, trows3, trows4)
    gsems = (gsem0, gsem1, gsem2, gsem3, gsem4)
    ssems = (ssem0, ssem1, ssem2, ssem3, ssem4)

    def gather_start(g, ci, b):
        off = (g // CPB) * LP + ci * C
        pltpu.make_async_copy(
            tok_tab.at[tix.at[pl.ds(off, C)]], trows[b], gsems[b]).start()

    def gather_wait(g, ci, b):
        off = (g // CPB) * LP + ci * C
        pltpu.make_async_copy(
            tok_tab.at[tix.at[pl.ds(off, C)]], trows[b], gsems[b]).wait()

    def scatter_start(g, ci, cw, b):
        pltpu.make_async_copy(
            trows[b].at[pl.ds(0, cw)],
            out.at[base_be + g // CPB, pl.ds(ci * C, cw)], ssems[b]).start()

    def scatter_wait(g, ci, cw, b):
        pltpu.make_async_copy(
            trows[b].at[pl.ds(0, cw)],
            out.at[base_be + g // CPB, pl.ds(ci * C, cw)], ssems[b]).wait()

    def chunk_body(g, k):
        ci, cw = _chunk_geom(k)
        b = k % NBUF
        gather_wait(g, ci, b)

        rbase = (g // CPB) * LP + ci * C

        @plsc.parallel_loop(0, cw, unroll=2)
        def _rows(r):
            v16 = pix[pl.ds(rbase + r, 16)]
            pb = v16[0] * D

            @plsc.parallel_loop(0, 2)
            def _half(q):
                for u in range(DL // 2):
                    o = q * (D // 2) + u * 16
                    plsc.addupdate(trows[b].at[r, pl.ds(o, 16)],
                                   ptab[pl.ds(pb + o, 16)])

        scatter_start(g, ci, cw, b)

        nb = (b + NBUF - 1) % NBUF
        ci_n, _ = _chunk_geom((k + NBUF - 1) % KSTEP)
        ci_p, cw_p = _chunk_geom((k + KSTEP - 1) % KSTEP)

        @pl.when(g + NBUF - 1 < NCHUNK)
        def _prefetch():
            @pl.when(g >= 1)
            def _wait_prev_scatter():
                scatter_wait(g - 1, ci_p, cw_p, nb)

            gather_start(g + NBUF - 1, ci_n, nb)

    for p in range(NBUF - 1):
        ci_p2, _ = _chunk_geom(p)
        gather_start(p, ci_p2, p)

    def outer(h, carry):
        for k in range(KSTEP):
            chunk_body(h * KSTEP + k, k)
        return carry

    lax.fori_loop(0, NCHUNK // KSTEP, outer, 0)

    for g in range(NCHUNK - NBUF, NCHUNK):
        k = g % KSTEP
        ci, cw = _chunk_geom(k)
        scatter_wait(g, ci, cw, k % NBUF)


@jax.jit
def _embed(tok80, pos80, token_table, position_table):
    mesh = plsc.VectorSubcoreMesh(core_axis_name="c", subcore_axis_name="s")
    k = functools.partial(
        pl.kernel,
        mesh=mesh,
        out_type=jax.ShapeDtypeStruct((B, L, D), jnp.float32),
        scratch_types=[
            pltpu.VMEM((IDS_W,), jnp.int32),
            pltpu.VMEM((IDS_W + 16,), jnp.int32),
            pltpu.VMEM((MAXLEN * D,), jnp.float32),
            pltpu.VMEM((C, D), jnp.float32),
            pltpu.VMEM((C, D), jnp.float32),
            pltpu.VMEM((C, D), jnp.float32),
            pltpu.VMEM((C, D), jnp.float32),
            pltpu.VMEM((C, D), jnp.float32),
            pltpu.SemaphoreType.DMA,
            pltpu.SemaphoreType.DMA,
            pltpu.SemaphoreType.DMA,
            pltpu.SemaphoreType.DMA,
            pltpu.SemaphoreType.DMA,
            pltpu.SemaphoreType.DMA,
            pltpu.SemaphoreType.DMA,
            pltpu.SemaphoreType.DMA,
            pltpu.SemaphoreType.DMA,
            pltpu.SemaphoreType.DMA,
        ],
    )(_sc_body)
    return k(tok80, pos80, token_table,
             jnp.reshape(position_table, (MAXLEN * D,)))


def _pad_ids(ids):
    ids = jnp.reshape(ids.astype(jnp.int32), (NW, BE_W, L))
    ids = jnp.pad(ids, ((0, 0), (0, 0), (0, LP - L)))
    return jnp.reshape(ids, (NW, IDS_W))


def kernel(input_ids, position_ids, token_table, position_table):
    return _embed(_pad_ids(input_ids), _pad_ids(position_ids),
                  token_table, position_table)


# speedup vs baseline: 1.0028x; 1.0028x over previous
import functools

import jax
import jax.numpy as jnp
from jax import lax
from jax.experimental import pallas as pl
from jax.experimental.pallas import tpu as pltpu
from jax.experimental.pallas import tpu_sc as plsc

B = 1024
L = 77
VOCAB = 49408
MAXLEN = 77
D = 1024

C = 8
CPB = 10
LP = C * CPB
CT = L - (CPB - 1) * C
NC = 2
NS = 16
NW = NC * NS
BE_W = B // NW
IDS_W = BE_W * LP
NCHUNK = BE_W * CPB
NBUF = 5
KSTEP = NBUF
DL = D // 16


def _sc_body(tok80, pos80, tok_tab, pos_tab, out,
             tix, pix, ptab, trows0, trows1, trows2, trows3, trows4,
             gsem0, gsem1, gsem2, gsem3, gsem4,
             ssem0, ssem1, ssem2, ssem3, ssem4):
    wid = lax.axis_index("s") * NC + lax.axis_index("c")
    base_be = wid * BE_W
    pltpu.sync_copy(tok80.at[wid], tix)
    pltpu.sync_copy(pos80.at[wid], pix.at[pl.ds(0, IDS_W)])
    pltpu.sync_copy(pos_tab, ptab)

    trows = (trows0, trows1, trows2, trows3, trows4)
    gsems = (gsem0, gsem1, gsem2, gsem3, gsem4)
    ssems = (ssem0, ssem1, ssem2, ssem3, ssem4)

    def gather_start(g, b):
        pltpu.make_async_copy(
            tok_tab.at[tix.at[pl.ds(g * C, C)]], trows[b], gsems[b]).start()

    def gather_wait(g, b):
        pltpu.make_async_copy(
            tok_tab.at[tix.at[pl.ds(g * C, C)]], trows[b], gsems[b]).wait()

    def _scatter(g, b, cw):
        if cw == C:
            off = pl.multiple_of((g % CPB) * C, C)
        else:
            off = (CPB - 1) * C
        return pltpu.make_async_copy(
            trows[b].at[pl.ds(0, cw)],
            out.at[base_be + g // CPB, pl.ds(off, cw)],
            ssems[b])

    def scatter_start(g, b):
        tail = g % CPB == CPB - 1

        @pl.when(jnp.logical_not(tail))
        def _full():
            _scatter(g, b, C).start()

        @pl.when(tail)
        def _tail():
            _scatter(g, b, CT).start()

    def scatter_wait(g, b):
        tail = g % CPB == CPB - 1

        @pl.when(jnp.logical_not(tail))
        def _full():
            _scatter(g, b, C).wait()

        @pl.when(tail)
        def _tail():
            _scatter(g, b, CT).wait()

    def chunk_body(g, b):
        gather_wait(g, b)
        rbase = g * C

        @plsc.parallel_loop(0, C, unroll=2)
        def _rows(r):
            v16 = pix[pl.ds(rbase + r, 16)]
            pid = v16[0]
            for j in range(DL):
                s = pl.ds(j * 16, 16)
                plsc.addupdate(trows[b].at[r, s], ptab[pid, s])

        scatter_start(g, b)

        nb = (b + NBUF - 1) % NBUF

        @pl.when(g + NBUF - 1 < NCHUNK)
        def _prefetch():
            @pl.when(g >= 1)
            def _wait_prev_scatter():
                scatter_wait(g - 1, nb)

            gather_start(g + NBUF - 1, nb)

    for p in range(NBUF - 1):
        gather_start(p, p)

    def outer(h, carry):
        for k in range(KSTEP):
            chunk_body(h * KSTEP + k, k)
        return carry

    lax.fori_loop(0, NCHUNK // KSTEP, outer, 0)

    for g in range(NCHUNK - NBUF, NCHUNK):
        b = g % NBUF
        cw = CT if g % CPB == CPB - 1 else C
        _scatter(g, b, cw).wait()


@jax.jit
def _embed(tok80, pos80, token_table, position_table):
    mesh = plsc.VectorSubcoreMesh(core_axis_name="c", subcore_axis_name="s")
    k = functools.partial(
        pl.kernel,
        mesh=mesh,
        out_type=jax.ShapeDtypeStruct((B, L, D), jnp.float32),
        scratch_types=[
            pltpu.VMEM((IDS_W,), jnp.int32),
            pltpu.VMEM((IDS_W + 16,), jnp.int32),
            pltpu.VMEM((MAXLEN, D), jnp.float32),
            pltpu.VMEM((C, D), jnp.float32),
            pltpu.VMEM((C, D), jnp.float32),
            pltpu.VMEM((C, D), jnp.float32),
            pltpu.VMEM((C, D), jnp.float32),
            pltpu.VMEM((C, D), jnp.float32),
            pltpu.SemaphoreType.DMA,
            pltpu.SemaphoreType.DMA,
            pltpu.SemaphoreType.DMA,
            pltpu.SemaphoreType.DMA,
            pltpu.SemaphoreType.DMA,
            pltpu.SemaphoreType.DMA,
            pltpu.SemaphoreType.DMA,
            pltpu.SemaphoreType.DMA,
            pltpu.SemaphoreType.DMA,
            pltpu.SemaphoreType.DMA,
        ],
    )(_sc_body)
    return k(tok80, pos80, token_table, position_table)


def _pad_ids(ids):
    ids = jnp.reshape(ids.astype(jnp.int32), (NW, BE_W, L))
    ids = jnp.pad(ids, ((0, 0), (0, 0), (0, LP - L)))
    return jnp.reshape(ids, (NW, IDS_W))


def kernel(input_ids, position_ids, token_table, position_table):
    return _embed(_pad_ids(input_ids), _pad_ids(position_ids),
                  token_table, position_table)


# speedup vs baseline: 1.0906x; 1.0876x over previous
import functools

import jax
import jax.numpy as jnp
from jax import lax
from jax.experimental import pallas as pl
from jax.experimental.pallas import tpu as pltpu
from jax.experimental.pallas import tpu_sc as plsc

B = 1024
L = 77
VOCAB = 49408
MAXLEN = 77
D = 1024

LP = 80
NC = 2
NS = 16
NW = NC * NS
BE_W = B // NW
IDS_W = BE_W * LP
C = 8
CPB = 10
NCHUNK = BE_W * CPB
NBUF = 5
KSTEP = 10
DL = D // 16


def _chunk_geom(k):
    ci = k % CPB
    cw = C if ci < CPB - 1 else L - (CPB - 1) * C
    return ci, cw


def _sc_body(tok80, pos80, tok_tab, pos_tab, out,
             tix, pix, ptab, trows0, trows1, trows2, trows3, trows4,
             gsem0, gsem1, gsem2, gsem3, gsem4,
             ssem0, ssem1, ssem2, ssem3, ssem4):
    wid = lax.axis_index("s") * NC + lax.axis_index("c")
    base_be = wid * BE_W
    pltpu.sync_copy(tok80.at[wid], tix)
    pltpu.sync_copy(pos80.at[wid], pix.at[pl.ds(0, IDS_W)])
    pltpu.sync_copy(pos_tab, ptab)

    trows = (trows0, trows1, trows2, trows3, trows4)
    gsems = (gsem0, gsem1, gsem2, gsem3, gsem4)
    ssems = (ssem0, ssem1, ssem2, ssem3, ssem4)

    def gather_start(g, ci, b):
        off = (g // CPB) * LP + ci * C
        pltpu.make_async_copy(
            tok_tab.at[tix.at[pl.ds(off, C)]], trows[b], gsems[b]).start()

    def gather_wait(g, ci, b):
        off = (g // CPB) * LP + ci * C
        pltpu.make_async_copy(
            tok_tab.at[tix.at[pl.ds(off, C)]], trows[b], gsems[b]).wait()

    def scatter_start(g, ci, cw, b):
        pltpu.make_async_copy(
            trows[b].at[pl.ds(0, cw)],
            out.at[base_be + g // CPB, pl.ds(ci * C, cw)], ssems[b]).start()

    def scatter_wait(g, ci, cw, b):
        pltpu.make_async_copy(
            trows[b].at[pl.ds(0, cw)],
            out.at[base_be + g // CPB, pl.ds(ci * C, cw)], ssems[b]).wait()

    def chunk_body(g, k):
        ci, cw = _chunk_geom(k)
        b = k % NBUF
        gather_wait(g, ci, b)

        rbase = (g // CPB) * LP + ci * C

        @plsc.parallel_loop(0, cw, unroll=1)
        def _rows(r):
            v16 = pix[pl.ds(rbase + r, 16)]
            pid = v16[0]
            for j in range(DL):
                s = pl.ds(j * 16, 16)
                plsc.addupdate(trows[b].at[r, s], ptab[pid, s])

        scatter_start(g, ci, cw, b)

        nb = (b + NBUF - 1) % NBUF
        ci_n, _ = _chunk_geom((k + NBUF - 1) % KSTEP)
        ci_p, cw_p = _chunk_geom((k + KSTEP - 1) % KSTEP)

        @pl.when(g + NBUF - 1 < NCHUNK)
        def _prefetch():
            @pl.when(g >= 1)
            def _wait_prev_scatter():
                scatter_wait(g - 1, ci_p, cw_p, nb)

            gather_start(g + NBUF - 1, ci_n, nb)

    for p in range(NBUF - 1):
        ci_p2, _ = _chunk_geom(p)
        gather_start(p, ci_p2, p)

    def outer(h, carry):
        for k in range(KSTEP):
            chunk_body(h * KSTEP + k, k)
        return carry

    lax.fori_loop(0, NCHUNK // KSTEP, outer, 0)

    for g in range(NCHUNK - NBUF, NCHUNK):
        k = g % KSTEP
        ci, cw = _chunk_geom(k)
        scatter_wait(g, ci, cw, k % NBUF)


@jax.jit
def _embed(tok80, pos80, token_table, position_table):
    mesh = plsc.VectorSubcoreMesh(core_axis_name="c", subcore_axis_name="s")
    k = functools.partial(
        pl.kernel,
        mesh=mesh,
        out_type=jax.ShapeDtypeStruct((B, L, D), jnp.float32),
        scratch_types=[
            pltpu.VMEM((IDS_W,), jnp.int32),
            pltpu.VMEM((IDS_W + 16,), jnp.int32),
            pltpu.VMEM((MAXLEN, D), jnp.float32),
            pltpu.VMEM((C, D), jnp.float32),
            pltpu.VMEM((C, D), jnp.float32),
            pltpu.VMEM((C, D), jnp.float32),
            pltpu.VMEM((C, D), jnp.float32),
            pltpu.VMEM((C, D), jnp.float32),
            pltpu.SemaphoreType.DMA,
            pltpu.SemaphoreType.DMA,
            pltpu.SemaphoreType.DMA,
            pltpu.SemaphoreType.DMA,
            pltpu.SemaphoreType.DMA,
            pltpu.SemaphoreType.DMA,
            pltpu.SemaphoreType.DMA,
            pltpu.SemaphoreType.DMA,
            pltpu.SemaphoreType.DMA,
            pltpu.SemaphoreType.DMA,
        ],
    )(_sc_body)
    return k(tok80, pos80, token_table, position_table)


def _pad_ids(ids):
    ids = jnp.reshape(ids.astype(jnp.int32), (NW, BE_W, L))
    ids = jnp.pad(ids, ((0, 0), (0, 0), (0, LP - L)))
    return jnp.reshape(ids, (NW, IDS_W))


def kernel(input_ids, position_ids, token_table, position_table):
    return _embed(_pad_ids(input_ids), _pad_ids(position_ids),
                  token_table, position_table)
